# Initial kernel scaffold; baseline (speedup 1.0000x reference)
import jax
import jax.numpy as jnp
from jax.experimental import pallas as pl


def kernel(x_nchw, stage_w, stage_gamma, stage_beta, stage_mean, stage_var, bott_w, bott_gamma, bott_beta, bott_mean, bott_var):
    raise NotImplementedError("write your pallas kernel here")


# speedup vs baseline: 3.7000x; 3.7000x over previous
import math
import functools
import numpy as np
import jax
import jax.numpy as jnp
from jax import lax
from jax.experimental import pallas as pl
from jax.experimental.pallas import tpu as pltpu


def _pool_mat(length, bins):
    m = np.zeros((bins, length), np.float32)
    for i in range(bins):
        s = (i * length) // bins
        e = -((-(i + 1) * length) // bins)
        m[i, s:e] = 1.0 / (e - s)
    return m


def _upsample_mat(l_out, l_in):
    u = np.zeros((l_out, l_in), np.float32)
    for o in range(l_out):
        src = max((o + 0.5) * l_in / l_out - 0.5, 0.0)
        i0 = min(int(math.floor(src)), l_in - 1)
        i1 = min(i0 + 1, l_in - 1)
        f = src - i0
        u[o, i0] += 1.0 - f
        u[o, i1] += f
    return u


def _bins_kernel(x_ref, pt_ref, w1_ref, mask_ref, b1m_ref, bins_ref, acc_ref):
    j = pl.program_id(1)

    @pl.when(j == 0)
    def _init():
        acc_ref[...] = jnp.zeros_like(acc_ref)

    acc_ref[...] += jnp.dot(x_ref[...], pt_ref[...],
                            preferred_element_type=jnp.float32)

    @pl.when(j == pl.num_programs(1) - 1)
    def _fin():
        z = jnp.dot(w1_ref[...], acc_ref[...].astype(w1_ref.dtype),
                    preferred_element_type=jnp.float32)
        bins_ref[...] = jnp.maximum(z * mask_ref[...] + b1m_ref[...], 0.0)


def _conv_kernel(xp_ref, w_ref, mix_ref, ut9_ref, out_ref, *, w_img, pad):
    k = pl.program_id(1)
    hw = out_ref.shape[-1]

    @pl.when(k == 0)
    def _init():
        out_ref[...] = jnp.dot(mix_ref[...], ut9_ref[...],
                               preferred_element_type=jnp.float32)

    col = lax.broadcasted_iota(jnp.int32, (1, hw), 1) % w_img
    acc = out_ref[...]
    for dy in range(3):
        for dx in range(3):
            start = pad + w_img * (dy - 1) + (dx - 1)
            s = xp_ref[:, start:start + hw]
            if dx == 0:
                s = s * (col != 0).astype(s.dtype)
            elif dx == 2:
                s = s * (col != w_img - 1).astype(s.dtype)
            acc = acc + jnp.dot(w_ref[3 * dy + dx], s,
                                preferred_element_type=jnp.float32)

    last = pl.num_programs(1) - 1

    @pl.when(k < last)
    def _store():
        out_ref[...] = acc

    @pl.when(k == last)
    def _store_relu():
        out_ref[...] = jnp.maximum(acc, 0.0)


def kernel(x_nchw, stage_w, stage_gamma, stage_beta, stage_mean, stage_var,
           bott_w, bott_gamma, bott_beta, bott_mean, bott_var):
    eps = 1e-5
    bin_sizes = (1, 2, 3, 6)
    cdt = jnp.bfloat16
    N, Cin, H, W = x_nchw.shape
    HW = H * W
    S = len(bin_sizes)
    Cs = stage_w.shape[1]
    SCs = S * Cs
    Cout = bott_w.shape[0]
    offs = np.concatenate([[0], np.cumsum([b * b for b in bin_sizes])]).astype(int)
    B2 = int(offs[-1])
    K2 = 9 * B2 + 1
    PAD = 2 * W
    Lp = HW + 2 * PAD

    pt = np.zeros((Lp, B2), np.float32)
    ut = np.zeros((B2, HW), np.float32)
    msk = np.zeros((SCs, B2), np.float32)
    for i, b in enumerate(bin_sizes):
        p2 = np.kron(_pool_mat(H, b), _pool_mat(W, b))
        u2 = np.kron(_upsample_mat(H, b), _upsample_mat(W, b))
        pt[PAD:PAD + HW, offs[i]:offs[i + 1]] = p2.T
        ut[offs[i]:offs[i + 1], :] = u2.T
        msk[i * Cs:(i + 1) * Cs, offs[i]:offs[i + 1]] = 1.0

    utp = np.zeros((B2, H + 2, W + 2), np.float32)
    utp[:, 1:-1, 1:-1] = ut.reshape(B2, H, W)
    ut9 = np.concatenate(
        [utp[:, dy:dy + H, dx:dx + W].reshape(B2, HW)
         for dy in range(3) for dx in range(3)]
        + [np.ones((1, HW), np.float32)], axis=0)

    sc_s = stage_gamma / jnp.sqrt(stage_var + eps)
    w1 = (stage_w * sc_s[:, :, None]).reshape(SCs, Cin).astype(cdt)
    b1 = (stage_beta - stage_mean * sc_s).reshape(SCs, 1)
    maskd = jnp.asarray(msk)
    b1m = (b1 * maskd).astype(jnp.float32)

    sc_b = bott_gamma / jnp.sqrt(bott_var + eps)
    w3 = bott_w * sc_b[:, None, None, None]
    b3 = bott_beta - bott_mean * sc_b
    w9 = jnp.transpose(w3[:, :Cin], (2, 3, 0, 1)).reshape(9, Cout, Cin).astype(cdt)
    w3s9 = jnp.transpose(w3[:, Cin:], (0, 2, 3, 1)).reshape(Cout, 9, SCs)

    xpf = jnp.pad(x_nchw.reshape(N, Cin, HW).astype(cdt),
                  ((0, 0), (0, 0), (PAD, PAD)))

    vmem = 64 * 1024 * 1024

    n_l = 2
    t_l = Lp // n_l
    bins = pl.pallas_call(
        _bins_kernel,
        out_shape=jax.ShapeDtypeStruct((N, SCs, B2), jnp.float32),
        grid_spec=pltpu.PrefetchScalarGridSpec(
            num_scalar_prefetch=0,
            grid=(N, n_l),
            in_specs=[
                pl.BlockSpec((None, Cin, t_l), lambda n, j: (n, 0, j)),
                pl.BlockSpec((t_l, B2), lambda n, j: (j, 0)),
                pl.BlockSpec((SCs, Cin), lambda n, j: (0, 0)),
                pl.BlockSpec((SCs, B2), lambda n, j: (0, 0)),
                pl.BlockSpec((SCs, B2), lambda n, j: (0, 0)),
            ],
            out_specs=pl.BlockSpec((None, SCs, B2), lambda n, j: (n, 0, 0)),
            scratch_shapes=[pltpu.VMEM((Cin, B2), jnp.float32)],
        ),
        compiler_params=pltpu.CompilerParams(
            dimension_semantics=("parallel", "arbitrary"),
            vmem_limit_bytes=vmem),
    )(xpf, jnp.asarray(pt).astype(cdt), w1, maskd, b1m)

    mix = jnp.einsum("ots,nsb->notb", w3s9, bins,
                     precision=lax.Precision.HIGHEST).reshape(N, Cout, 9 * B2)
    mix = jnp.concatenate(
        [mix, jnp.broadcast_to(b3.reshape(1, Cout, 1), (N, Cout, 1))],
        axis=2).astype(cdt)

    n_k = 4
    Ckb = Cin // n_k
    out = pl.pallas_call(
        functools.partial(_conv_kernel, w_img=W, pad=PAD),
        out_shape=jax.ShapeDtypeStruct((N, Cout, HW), jnp.float32),
        grid_spec=pltpu.PrefetchScalarGridSpec(
            num_scalar_prefetch=0,
            grid=(N, n_k),
            in_specs=[
                pl.BlockSpec((None, Ckb, Lp), lambda n, k: (n, k, 0)),
                pl.BlockSpec((9, Cout, Ckb), lambda n, k: (0, 0, k)),
                pl.BlockSpec((None, Cout, K2), lambda n, k: (n, 0, 0)),
                pl.BlockSpec((K2, HW), lambda n, k: (0, 0)),
            ],
            out_specs=pl.BlockSpec((None, Cout, HW), lambda n, k: (n, 0, 0)),
        ),
        compiler_params=pltpu.CompilerParams(
            dimension_semantics=("parallel", "arbitrary"),
            vmem_limit_bytes=vmem),
    )(xpf, w9, mix, jnp.asarray(ut9).astype(cdt))

    return out.reshape(N, Cout, H, W)


# speedup vs baseline: 4.7981x; 1.2968x over previous
import math
import functools
import numpy as np
import jax
import jax.numpy as jnp
from jax import lax
from jax.experimental import pallas as pl
from jax.experimental.pallas import tpu as pltpu


def _pool_mat(length, bins):
    m = np.zeros((bins, length), np.float32)
    for i in range(bins):
        s = (i * length) // bins
        e = -((-(i + 1) * length) // bins)
        m[i, s:e] = 1.0 / (e - s)
    return m


def _upsample_mat(l_out, l_in):
    u = np.zeros((l_out, l_in), np.float32)
    for o in range(l_out):
        src = max((o + 0.5) * l_in / l_out - 0.5, 0.0)
        i0 = min(int(math.floor(src)), l_in - 1)
        i1 = min(i0 + 1, l_in - 1)
        f = src - i0
        u[o, i0] += 1.0 - f
        u[o, i1] += f
    return u


def _bins_kernel(x_ref, pt_ref, w1_ref, mask_ref, b1m_ref, bins_ref, acc_ref):
    j = pl.program_id(1)

    @pl.when(j == 0)
    def _init():
        acc_ref[...] = jnp.zeros_like(acc_ref)

    acc_ref[...] += jnp.dot(x_ref[...].astype(pt_ref.dtype), pt_ref[...],
                            preferred_element_type=jnp.float32)

    @pl.when(j == pl.num_programs(1) - 1)
    def _fin():
        z = jnp.dot(w1_ref[...], acc_ref[...].astype(w1_ref.dtype),
                    preferred_element_type=jnp.float32)
        bins_ref[...] = jnp.maximum(z * mask_ref[...] + b1m_ref[...], 0.0)


def _conv_kernel(x_ref, w_ref, mix_ref, ut9_ref, out_ref, xs_ref, *, w_img, pad):
    k = pl.program_id(1)
    hw = x_ref.shape[-1]

    @pl.when(k == 0)
    def _init():
        out_ref[...] = jnp.dot(mix_ref[...], ut9_ref[...],
                               preferred_element_type=jnp.float32)

    @pl.when((pl.program_id(0) == 0) & (k == 0))
    def _zero_halo():
        xs_ref[:, :pad] = jnp.zeros_like(xs_ref[:, :pad])
        xs_ref[:, pad + hw:] = jnp.zeros_like(xs_ref[:, pad + hw:])

    xs_ref[:, pad:pad + hw] = x_ref[...].astype(xs_ref.dtype)

    col = lax.broadcasted_iota(jnp.int32, (1, hw), 1) % w_img
    acc = out_ref[...]
    for dy in range(3):
        for dx in range(3):
            start = pad + w_img * (dy - 1) + (dx - 1)
            s = xs_ref[:, start:start + hw]
            if dx == 0:
                s = s * (col != 0).astype(s.dtype)
            elif dx == 2:
                s = s * (col != w_img - 1).astype(s.dtype)
            acc = acc + jnp.dot(w_ref[3 * dy + dx], s,
                                preferred_element_type=jnp.float32)

    last = pl.num_programs(1) - 1

    @pl.when(k < last)
    def _store():
        out_ref[...] = acc

    @pl.when(k == last)
    def _store_relu():
        out_ref[...] = jnp.maximum(acc, 0.0)


def kernel(x_nchw, stage_w, stage_gamma, stage_beta, stage_mean, stage_var,
           bott_w, bott_gamma, bott_beta, bott_mean, bott_var):
    eps = 1e-5
    bin_sizes = (1, 2, 3, 6)
    cdt = jnp.bfloat16
    N, Cin, H, W = x_nchw.shape
    HW = H * W
    S = len(bin_sizes)
    Cs = stage_w.shape[1]
    SCs = S * Cs
    Cout = bott_w.shape[0]
    offs = np.concatenate([[0], np.cumsum([b * b for b in bin_sizes])]).astype(int)
    B2 = int(offs[-1])
    K2 = 9 * B2 + 1
    PAD = 2 * W

    pt = np.zeros((HW, B2), np.float32)
    ut = np.zeros((B2, HW), np.float32)
    msk = np.zeros((SCs, B2), np.float32)
    for i, b in enumerate(bin_sizes):
        p2 = np.kron(_pool_mat(H, b), _pool_mat(W, b))
        u2 = np.kron(_upsample_mat(H, b), _upsample_mat(W, b))
        pt[:, offs[i]:offs[i + 1]] = p2.T
        ut[offs[i]:offs[i + 1], :] = u2.T
        msk[i * Cs:(i + 1) * Cs, offs[i]:offs[i + 1]] = 1.0

    utp = np.zeros((B2, H + 2, W + 2), np.float32)
    utp[:, 1:-1, 1:-1] = ut.reshape(B2, H, W)
    ut9 = np.concatenate(
        [utp[:, dy:dy + H, dx:dx + W].reshape(B2, HW)
         for dy in range(3) for dx in range(3)]
        + [np.ones((1, HW), np.float32)], axis=0)

    sc_s = stage_gamma / jnp.sqrt(stage_var + eps)
    w1 = (stage_w * sc_s[:, :, None]).reshape(SCs, Cin).astype(cdt)
    b1 = (stage_beta - stage_mean * sc_s).reshape(SCs, 1)
    maskd = jnp.asarray(msk)
    b1m = (b1 * maskd).astype(jnp.float32)

    sc_b = bott_gamma / jnp.sqrt(bott_var + eps)
    w3 = bott_w * sc_b[:, None, None, None]
    b3 = bott_beta - bott_mean * sc_b
    w9 = jnp.transpose(w3[:, :Cin], (2, 3, 0, 1)).reshape(9, Cout, Cin).astype(cdt)
    w3s9 = jnp.transpose(w3[:, Cin:], (0, 2, 3, 1)).reshape(Cout, 9, SCs)

    x_cm = x_nchw.reshape(N, Cin, HW)
    vmem = 64 * 1024 * 1024

    n_l = 4
    t_hw = HW // n_l
    bins = pl.pallas_call(
        _bins_kernel,
        out_shape=jax.ShapeDtypeStruct((N, SCs, B2), jnp.float32),
        grid_spec=pltpu.PrefetchScalarGridSpec(
            num_scalar_prefetch=0,
            grid=(N, n_l),
            in_specs=[
                pl.BlockSpec((None, Cin, t_hw), lambda n, j: (n, 0, j)),
                pl.BlockSpec((t_hw, B2), lambda n, j: (j, 0)),
                pl.BlockSpec((SCs, Cin), lambda n, j: (0, 0)),
                pl.BlockSpec((SCs, B2), lambda n, j: (0, 0)),
                pl.BlockSpec((SCs, B2), lambda n, j: (0, 0)),
            ],
            out_specs=pl.BlockSpec((None, SCs, B2), lambda n, j: (n, 0, 0)),
            scratch_shapes=[pltpu.VMEM((Cin, B2), jnp.float32)],
        ),
        compiler_params=pltpu.CompilerParams(
            dimension_semantics=("parallel", "arbitrary"),
            vmem_limit_bytes=vmem),
    )(x_cm, jnp.asarray(pt).astype(cdt), w1, maskd, b1m)

    mix = jnp.einsum("ots,nsb->notb", w3s9.astype(cdt), bins.astype(cdt),
                     preferred_element_type=jnp.float32).reshape(N, Cout, 9 * B2)
    mix = jnp.concatenate(
        [mix, jnp.broadcast_to(b3.reshape(1, Cout, 1), (N, Cout, 1))],
        axis=2).astype(cdt)

    n_k = 8
    Ckb = Cin // n_k
    out = pl.pallas_call(
        functools.partial(_conv_kernel, w_img=W, pad=PAD),
        out_shape=jax.ShapeDtypeStruct((N, Cout, HW), jnp.float32),
        grid_spec=pltpu.PrefetchScalarGridSpec(
            num_scalar_prefetch=0,
            grid=(N, n_k),
            in_specs=[
                pl.BlockSpec((None, Ckb, HW), lambda n, k: (n, k, 0)),
                pl.BlockSpec((9, Cout, Ckb), lambda n, k: (0, 0, k)),
                pl.BlockSpec((None, Cout, K2), lambda n, k: (n, 0, 0)),
                pl.BlockSpec((K2, HW), lambda n, k: (0, 0)),
            ],
            out_specs=pl.BlockSpec((None, Cout, HW), lambda n, k: (n, 0, 0)),
            scratch_shapes=[pltpu.VMEM((Ckb, HW + 2 * PAD), cdt)],
        ),
        compiler_params=pltpu.CompilerParams(
            dimension_semantics=("parallel", "arbitrary"),
            vmem_limit_bytes=vmem),
    )(x_cm, w9, mix, jnp.asarray(ut9).astype(cdt))

    return out.reshape(N, Cout, H, W)
